# baseline (speedup 1.0000x reference)
import functools

import jax
import jax.numpy as jnp
from jax import lax
from jax.experimental import pallas as pl
from jax.experimental.pallas import tpu as pltpu
from jax.experimental.pallas import tpu_sc as plsc

T = 2048
G = 16
NLANES = 16
NWORKERS = 32
CUNROLL = 16


def _index_stage(len_ref, g0_ref, g1_ref, w_ref, *, lmax):
    B = len_ref.shape[0]
    L = len_ref[...]
    Lf = L.astype(jnp.float32)
    j = lax.broadcasted_iota(jnp.int32, (B, T), 1).astype(jnp.float32)
    scale = Lf * (1.0 / float(T))
    src = (j + 0.5) * scale - 0.5
    src = jnp.clip(src, 0.0, jnp.maximum(Lf - 1.0, 0.0))
    i0 = src.astype(jnp.int32)
    i1 = jnp.minimum(i0 + 1, L - 1)
    w = src - i0.astype(jnp.float32)
    roff = lax.broadcasted_iota(jnp.int32, (B, T), 0) * lmax
    g0_ref[...] = roff + i0
    g1_ref[...] = roff + i1
    w_ref[...] = jnp.broadcast_to(w[:, :, None], (B, T, NLANES))


def _build_indices(lengths, B, Lmax):
    return pl.pallas_call(
        functools.partial(_index_stage, lmax=Lmax),
        out_shape=[
            jax.ShapeDtypeStruct((B, T), jnp.int32),
            jax.ShapeDtypeStruct((B, T), jnp.int32),
            jax.ShapeDtypeStruct((B, T, NLANES), jnp.float32),
        ],
    )(lengths.reshape(B, 1))


def _make_sc_resample(N, C):
    mesh = plsc.VectorSubcoreMesh(core_axis_name="c", subcore_axis_name="s")
    rows_per_worker = N // NWORKERS
    chpw = rows_per_worker // G

    @functools.partial(
        pl.kernel,
        mesh=mesh,
        out_type=jax.ShapeDtypeStruct((N, C), jnp.float32),
        scratch_types=[
            pltpu.VMEM((chpw * G // 128, 128), jnp.int32),
            pltpu.VMEM((chpw * G // 128, 128), jnp.int32),
            pltpu.VMEM((rows_per_worker * NLANES // 128, 128),
                       jnp.float32),
            pltpu.VMEM((G, C), jnp.float32),
            pltpu.VMEM((G, C), jnp.float32),
            pltpu.VMEM((G, C), jnp.float32),
            pltpu.VMEM((G, C), jnp.float32),
            pltpu.SemaphoreType.DMA,
            pltpu.SemaphoreType.DMA,
            pltpu.SemaphoreType.DMA,
            pltpu.SemaphoreType.DMA,
            pltpu.SemaphoreType.DMA,
            pltpu.SemaphoreType.DMA,
        ],
    )
    def sc_resample(x_hbm, g0_hbm, g1_hbm, w_hbm, out_hbm,
                    g0_all, g1_all, w_all,
                    r0_a, r0_b, r1_a, r1_b,
                    sg0_a, sg0_b, sg1_a, sg1_b, so_a, so_b):
        r0v = (r0_a, r0_b)
        r1v = (r1_a, r1_b)
        sg0 = (sg0_a, sg0_b)
        sg1 = (sg1_a, sg1_b)
        so = (so_a, so_b)

        wid = lax.axis_index("s") * 2 + lax.axis_index("c")
        row0 = wid * rows_per_worker
        gidx_rows = chpw * G // 128
        w_rows = rows_per_worker * NLANES // 128

        pltpu.sync_copy(g0_hbm.at[pl.ds(wid * gidx_rows, gidx_rows), :], g0_all)
        pltpu.sync_copy(g1_hbm.at[pl.ds(wid * gidx_rows, gidx_rows), :], g1_all)
        pltpu.sync_copy(w_hbm.at[pl.ds(wid * w_rows, w_rows), :], w_all)

        def idx_ref(all_ref, k):
            return all_ref.at[k // 8, pl.ds((k % 8) * G, G)]

        def fire_gathers(k, s):
            pltpu.make_async_copy(x_hbm.at[idx_ref(g0_all, k)], r0v[s],
                                  sg0[s]).start()
            pltpu.make_async_copy(x_hbm.at[idx_ref(g1_all, k)], r1v[s],
                                  sg1[s]).start()

        def wait_gathers(k, s):
            pltpu.make_async_copy(x_hbm.at[idx_ref(g0_all, k)], r0v[s],
                                  sg0[s]).wait()
            pltpu.make_async_copy(x_hbm.at[idx_ref(g1_all, k)], r1v[s],
                                  sg1[s]).wait()

        def out_copy(k, s):
            return pltpu.make_async_copy(
                r0v[s], out_hbm.at[pl.ds(row0 + k * G, G), :], so[s])

        fire_gathers(0, 0)

        @pl.loop(0, chpw, step=2)
        def _(k0):
            for b in range(2):
                k = k0 + b
                s, ns = b, 1 - b

                @pl.when(k >= 1)
                def _():
                    out_copy(k, ns).wait()

                @pl.when(k + 1 < chpw)
                def _():
                    fire_gathers(k + 1, ns)

                wait_gathers(k, s)

                for r in range(G):
                    wv = w_all[pl.ds(2 * k + r // 8, 1),
                               pl.ds((r % 8) * NLANES, NLANES)]

                    @pl.loop(0, C, step=NLANES * CUNROLL)
                    def _(cc):
                        for u in range(CUNROLL):
                            sl = (pl.ds(r, 1), pl.ds(cc + u * NLANES, NLANES))
                            a = r0v[s][sl]
                            bb = r1v[s][sl]
                            r0v[s][sl] = a + wv * (bb - a)

                out_copy(k, s).start()

        out_copy(chpw - 1, 1).wait()

    return sc_resample


def kernel(x, lengths):
    B, Lmax, C = x.shape
    N = B * T
    g0, g1, w = _build_indices(lengths, B, Lmax)
    x2 = x.reshape(B * Lmax, C)
    out2 = _make_sc_resample(N, C)(
        x2,
        g0.reshape(N // 128, 128),
        g1.reshape(N // 128, 128),
        w.reshape(N * NLANES // 128, 128),
    )
    return out2.reshape(B, T, C)


# speedup vs baseline: 1.0797x; 1.0797x over previous
import functools

import jax
import jax.numpy as jnp
from jax import lax
from jax.experimental import pallas as pl
from jax.experimental.pallas import tpu as pltpu
from jax.experimental.pallas import tpu_sc as plsc

T = 2048
G = 16
NLANES = 16
NWORKERS = 32
CUNROLL = 8


def _index_stage(len_ref, g0_ref, g1_ref, w_ref, *, lmax):
    B = len_ref.shape[0]
    L = len_ref[...]
    Lf = L.astype(jnp.float32)
    j = lax.broadcasted_iota(jnp.int32, (B, T), 1).astype(jnp.float32)
    scale = Lf * (1.0 / float(T))
    src = (j + 0.5) * scale - 0.5
    src = jnp.clip(src, 0.0, jnp.maximum(Lf - 1.0, 0.0))
    i0 = src.astype(jnp.int32)
    i1 = jnp.minimum(i0 + 1, L - 1)
    w = src - i0.astype(jnp.float32)
    roff = lax.broadcasted_iota(jnp.int32, (B, T), 0) * lmax
    g0_ref[...] = roff + i0
    g1_ref[...] = roff + i1
    w_ref[...] = jnp.broadcast_to(w[:, :, None], (B, T, NLANES))


def _build_indices(lengths, B, Lmax):
    return pl.pallas_call(
        functools.partial(_index_stage, lmax=Lmax),
        out_shape=[
            jax.ShapeDtypeStruct((B, T), jnp.int32),
            jax.ShapeDtypeStruct((B, T), jnp.int32),
            jax.ShapeDtypeStruct((B, T, NLANES), jnp.float32),
        ],
    )(lengths.reshape(B, 1))


def _make_sc_resample(N, C):
    mesh = plsc.VectorSubcoreMesh(core_axis_name="c", subcore_axis_name="s")
    rows_per_worker = N // NWORKERS
    chpw = rows_per_worker // G

    @functools.partial(
        pl.kernel,
        mesh=mesh,
        out_type=jax.ShapeDtypeStruct((N, C), jnp.float32),
        scratch_types=[
            pltpu.VMEM((chpw * G // 128, 128), jnp.int32),
            pltpu.VMEM((chpw * G // 128, 128), jnp.int32),
            pltpu.VMEM((rows_per_worker * NLANES // 128, 128),
                       jnp.float32),
            pltpu.VMEM((G, C), jnp.float32),
            pltpu.VMEM((G, C), jnp.float32),
            pltpu.VMEM((G, C), jnp.float32),
            pltpu.VMEM((G, C), jnp.float32),
            pltpu.SemaphoreType.DMA,
            pltpu.SemaphoreType.DMA,
            pltpu.SemaphoreType.DMA,
            pltpu.SemaphoreType.DMA,
            pltpu.SemaphoreType.DMA,
            pltpu.SemaphoreType.DMA,
        ],
    )
    def sc_resample(x_hbm, g0_hbm, g1_hbm, w_hbm, out_hbm,
                    g0_all, g1_all, w_all,
                    r0_a, r0_b, r1_a, r1_b,
                    sg0_a, sg0_b, sg1_a, sg1_b, so_a, so_b):
        r0v = (r0_a, r0_b)
        r1v = (r1_a, r1_b)
        sg0 = (sg0_a, sg0_b)
        sg1 = (sg1_a, sg1_b)
        so = (so_a, so_b)

        wid = lax.axis_index("s") * 2 + lax.axis_index("c")
        row0 = wid * rows_per_worker
        gidx_rows = chpw * G // 128
        w_rows = rows_per_worker * NLANES // 128

        pltpu.sync_copy(g0_hbm.at[pl.ds(wid * gidx_rows, gidx_rows), :], g0_all)
        pltpu.sync_copy(g1_hbm.at[pl.ds(wid * gidx_rows, gidx_rows), :], g1_all)
        pltpu.sync_copy(w_hbm.at[pl.ds(wid * w_rows, w_rows), :], w_all)

        def idx_ref(all_ref, k):
            return all_ref.at[k // 8, pl.ds((k % 8) * G, G)]

        def fire_gathers(k, s):
            pltpu.make_async_copy(x_hbm.at[idx_ref(g0_all, k)], r0v[s],
                                  sg0[s]).start()
            pltpu.make_async_copy(x_hbm.at[idx_ref(g1_all, k)], r1v[s],
                                  sg1[s]).start()

        def wait_gathers(k, s):
            pltpu.make_async_copy(x_hbm.at[idx_ref(g0_all, k)], r0v[s],
                                  sg0[s]).wait()
            pltpu.make_async_copy(x_hbm.at[idx_ref(g1_all, k)], r1v[s],
                                  sg1[s]).wait()

        def out_copy(k, s):
            return pltpu.make_async_copy(
                r0v[s], out_hbm.at[pl.ds(row0 + k * G, G), :], so[s])

        fire_gathers(0, 0)

        @pl.loop(0, chpw, step=2)
        def _(k0):
            for b in range(2):
                k = k0 + b
                s, ns = b, 1 - b

                @pl.when(k >= 1)
                def _():
                    out_copy(k, ns).wait()

                @pl.when(k + 1 < chpw)
                def _():
                    fire_gathers(k + 1, ns)

                wait_gathers(k, s)

                for r in range(G):
                    wv = w_all[pl.ds(2 * k + r // 8, 1),
                               pl.ds((r % 8) * NLANES, NLANES)]

                    @pl.loop(0, C, step=NLANES * CUNROLL)
                    def _(cc):
                        for u in range(CUNROLL):
                            sl = (pl.ds(r, 1), pl.ds(cc + u * NLANES, NLANES))
                            a = r0v[s][sl]
                            bb = r1v[s][sl]
                            r0v[s][sl] = a + wv * (bb - a)

                out_copy(k, s).start()

        out_copy(chpw - 1, 1).wait()

    return sc_resample


def kernel(x, lengths):
    B, Lmax, C = x.shape
    N = B * T
    g0, g1, w = _build_indices(lengths, B, Lmax)
    x2 = x.reshape(B * Lmax, C)
    out2 = _make_sc_resample(N, C)(
        x2,
        g0.reshape(N // 128, 128),
        g1.reshape(N // 128, 128),
        w.reshape(N * NLANES // 128, 128),
    )
    return out2.reshape(B, T, C)


# speedup vs baseline: 1.2085x; 1.1192x over previous
import dataclasses
import functools

import jax
import jax.numpy as jnp
from jax import lax
from jax.experimental import pallas as pl
from jax.experimental.pallas import tpu as pltpu
from jax.experimental.pallas import tpu_sc as plsc

T = 2048
G = 16
NLANES = 16
NWORKERS = 32
CUNROLL = 8


def _splat(vec, i):
    idx = jnp.full((NLANES, 1), i, dtype=jnp.int32)
    dn = lax.GatherDimensionNumbers(
        offset_dims=(), collapsed_slice_dims=(0,), start_index_map=(0,))
    return lax.gather(vec, idx, dn, slice_sizes=(1,),
                      mode=lax.GatherScatterMode.PROMISE_IN_BOUNDS)


def _make_sc_resample(B, Lmax, C):
    N = B * T
    mesh = plsc.VectorSubcoreMesh(core_axis_name="c", subcore_axis_name="s")
    rows_per_worker = N // NWORKERS
    chpw = rows_per_worker // G
    wpb = NWORKERS // B

    cp = pltpu.CompilerParams()
    if "needs_layout_passes" in pltpu.CompilerParams.__dataclass_fields__:
        cp = dataclasses.replace(cp, needs_layout_passes=False)

    @functools.partial(
        pl.kernel,
        mesh=mesh,
        compiler_params=cp,
        out_type=jax.ShapeDtypeStruct((N, C), jnp.float32),
        scratch_types=[
            pltpu.VMEM((1, NLANES), jnp.int32),
            pltpu.VMEM((2 * G, C), jnp.float32),
            pltpu.VMEM((2 * G, C), jnp.float32),
            pltpu.VMEM((G, C), jnp.float32),
            pltpu.VMEM((G, C), jnp.float32),
            pltpu.SemaphoreType.DMA,
            pltpu.SemaphoreType.DMA,
            pltpu.SemaphoreType.DMA,
            pltpu.SemaphoreType.DMA,
            pltpu.SemaphoreType.DMA,
            pltpu.SemaphoreType.DMA,
        ],
    )
    def sc_resample(x_hbm, l_hbm, out_hbm,
                    lv, win_a, win_b, o_a, o_b,
                    sg0_a, sg0_b, sg1_a, sg1_b, so_a, so_b):
        win = (win_a, win_b)
        ov = (o_a, o_b)
        sg0 = (sg0_a, sg0_b)
        sg1 = (sg1_a, sg1_b)
        so = (so_a, so_b)

        wid = lax.axis_index("s") * 2 + lax.axis_index("c")
        row0 = wid * rows_per_worker

        pltpu.sync_copy(l_hbm, lv)
        lvv = lv[0, :]

        iota = lax.iota(jnp.int32, NLANES)
        iota_f = iota.astype(jnp.float32)

        def chunk_math(bb, k):
            L = _splat(lvv, bb)
            Lf = L.astype(jnp.float32)
            scale = Lf * (1.0 / float(T))
            j0 = (wid % wpb) * rows_per_worker + k * G
            jv = j0.astype(jnp.float32) + iota_f
            src = (jv + 0.5) * scale - 0.5
            src = jnp.minimum(jnp.maximum(src, 0.0), Lf - 1.0)
            i0 = src.astype(jnp.int32)
            i1 = jnp.minimum(i0 + 1, L - 1)
            w = src - i0.astype(jnp.float32)
            return i0, i1, w

        def fire(bb, roff, k, s):
            i0, i1, _ = chunk_math(bb, k)
            pltpu.make_async_copy(x_hbm.at[roff + i0],
                                  win[s].at[pl.ds(0, G), :], sg0[s]).start()
            pltpu.make_async_copy(x_hbm.at[roff + i1],
                                  win[s].at[pl.ds(G, G), :], sg1[s]).start()

        def wait_fill(s):
            pltpu.make_async_copy(x_hbm.at[iota],
                                  win[s].at[pl.ds(0, G), :], sg0[s]).wait()
            pltpu.make_async_copy(x_hbm.at[iota],
                                  win[s].at[pl.ds(G, G), :], sg1[s]).wait()

        def out_copy(k, s):
            return pltpu.make_async_copy(
                ov[s], out_hbm.at[pl.ds(row0 + k * G, G), :], so[s])

        bb = wid // wpb
        roff = bb * Lmax

        fire(bb, roff, 0, 0)

        @pl.loop(0, chpw, step=2)
        def _(k0):
            for slot in range(2):
                k = k0 + slot
                s, ns = slot, 1 - slot

                @pl.when(k + 1 < chpw)
                def _():
                    fire(bb, roff, k + 1, ns)

                wait_fill(s)

                @pl.when(k >= 2)
                def _():
                    out_copy(k, s).wait()

                _, _, w = chunk_math(bb, k)
                for r in range(G):
                    wspl = _splat(w, r)

                    @pl.loop(0, C, step=NLANES * CUNROLL)
                    def _(cc):
                        for u in range(CUNROLL):
                            sl = pl.ds(cc + u * NLANES, NLANES)
                            a = win[s][r, sl]
                            b2 = win[s][G + r, sl]
                            ov[s][r, sl] = a + wspl * (b2 - a)

                out_copy(k, s).start()

        out_copy(chpw - 2, 0).wait()
        out_copy(chpw - 1, 1).wait()

    return sc_resample


def kernel(x, lengths):
    B, Lmax, C = x.shape
    x2 = x.reshape(B * Lmax, C)
    lp = jnp.pad(lengths, (0, NLANES - B)).reshape(1, NLANES)
    out2 = _make_sc_resample(B, Lmax, C)(x2, lp)
    return out2.reshape(B, T, C)
